# baseline (speedup 1.0000x reference)
import functools

import jax
import jax.numpy as jnp
from jax import lax
from jax.experimental import pallas as pl
from jax.experimental.pallas import tpu as pltpu
from jax.experimental.pallas import tpu_sc as plsc

DIM = 1024
HIDDEN = 2048
E = 16
N_TOK = 2048
CAP = 160
SLOTS = E * CAP

NC = 2
NS = 16
NW = NC * NS
DISP_CHUNK = SLOTS // NW
COMB_CHUNK = N_TOK // NW

HB = 1024
NH = HIDDEN // HB


def _router_body(x_ref, gw_ref, dsc_ref, dg_ref, wp_ref):
    x = x_ref[...]
    gw = gw_ref[...]
    logits = lax.dot_general(
        x, gw, (((1,), (1,)), ((), ())),
        preferred_element_type=jnp.float32,
        precision=lax.Precision.DEFAULT)

    m = jnp.max(logits, axis=1, keepdims=True)
    ex = jnp.exp(logits - m)
    s = jnp.sum(ex, axis=1, keepdims=True)
    p = 1.0 / s
    w = p / (p + 1e-9)

    lane = lax.broadcasted_iota(jnp.int32, (N_TOK, E), 1)
    ids = jnp.min(jnp.where(logits == m, lane, E), axis=1, keepdims=True)

    onehot = (lane == ids).astype(jnp.float32)

    rows = lax.broadcasted_iota(jnp.int32, (N_TOK, E), 0)
    cum = onehot
    k = 1
    while k < N_TOK:
        rolled = pltpu.roll(cum, k, 0)
        cum = cum + jnp.where(rows >= k, rolled, 0.0)
        k *= 2

    slot = (jnp.sum(onehot * cum, axis=1, keepdims=True) - 1.0).astype(jnp.int32)

    cnt = cum[N_TOK - 1:N_TOK, :].astype(jnp.int32)
    lane1 = lax.broadcasted_iota(jnp.int32, (1, E), 1)
    e_star = jnp.min(jnp.where(cnt < CAP, lane1, E), axis=1, keepdims=True)
    cnt_star = jnp.sum(jnp.where(lane1 == e_star, cnt, 0), axis=1, keepdims=True)
    sentinel = e_star * CAP + cnt_star

    valid = slot < CAP
    dest = ids * CAP + slot
    dsc_ref[...] = jnp.where(valid, dest, SLOTS)
    dg_ref[...] = jnp.where(valid, dest, jnp.broadcast_to(sentinel, (N_TOK, 1)))
    wp_ref[...] = w


def _router(x_flat, gate_W):
    return pl.pallas_call(
        _router_body,
        out_shape=(
            jax.ShapeDtypeStruct((N_TOK, 1), jnp.int32),
            jax.ShapeDtypeStruct((N_TOK, 1), jnp.int32),
            jax.ShapeDtypeStruct((N_TOK, 1), jnp.float32),
        ),
    )(x_flat, gate_W)


def _dispatch_body(x_hbm, dsc_hbm, wp_hbm, xe_hbm, ws_hbm,
                   dsc_v, wp_v, tidx_v, wslot_v, idx_v, wsb_v, rows_v, sem):
    wid = lax.axis_index("s") * NC + lax.axis_index("c")
    pltpu.sync_copy(dsc_hbm, dsc_v)
    pltpu.sync_copy(wp_hbm, wp_v)

    zero_f = jnp.zeros((16,), jnp.float32)
    lane = lax.iota(jnp.int32, 16)

    @plsc.parallel_loop(0, SLOTS // 16, unroll=8)
    def _init(i):
        off = pl.multiple_of(i * 16, 16)
        p = lane + i * 16
        tidx_v[pl.ds(off, 16)] = jnp.where(p < N_TOK, p, p - N_TOK)
        wslot_v[pl.ds(off, 16)] = zero_f

    @plsc.parallel_loop(0, N_TOK // 16, unroll=8)
    def _scatter(i):
        off = pl.multiple_of(i * 16, 16)
        d = dsc_v[pl.ds(off, 16)]
        mvalid = d < SLOTS
        dc = jnp.minimum(d, SLOTS - 1)
        plsc.store_scatter(tidx_v, [dc], lane + i * 16, mask=mvalid)
        wv = wp_v[pl.ds(off, 16)]
        plsc.store_scatter(wslot_v, [dc], wv, mask=mvalid)

    base = wid * DISP_CHUNK
    for j in range(DISP_CHUNK // 16):
        src = pl.multiple_of(base + j * 16, 16)
        idx_v[pl.ds(j * 16, 16)] = tidx_v[pl.ds(src, 16)]
        wsb_v[pl.ds(j * 16, 16)] = wslot_v[pl.ds(src, 16)]

    pltpu.async_copy(x_hbm.at[idx_v], rows_v, sem).wait()
    pltpu.sync_copy(rows_v, xe_hbm.at[pl.ds(base, DISP_CHUNK)])
    pltpu.sync_copy(wsb_v, ws_hbm.at[pl.ds(base, DISP_CHUNK)])


def _dispatch(x_flat, dsc, wp):
    mesh = plsc.VectorSubcoreMesh(core_axis_name="c", subcore_axis_name="s")
    f = pl.kernel(
        _dispatch_body,
        out_type=(
            jax.ShapeDtypeStruct((SLOTS, DIM), jnp.float32),
            jax.ShapeDtypeStruct((SLOTS,), jnp.float32),
        ),
        mesh=mesh,
        scratch_types=[
            pltpu.VMEM((N_TOK,), jnp.int32),
            pltpu.VMEM((N_TOK,), jnp.float32),
            pltpu.VMEM((SLOTS,), jnp.int32),
            pltpu.VMEM((SLOTS,), jnp.float32),
            pltpu.VMEM((DISP_CHUNK,), jnp.int32),
            pltpu.VMEM((DISP_CHUNK,), jnp.float32),
            pltpu.VMEM((DISP_CHUNK, DIM), jnp.float32),
            pltpu.SemaphoreType.DMA,
        ],
        compiler_params=pltpu.CompilerParams(needs_layout_passes=False,
                                             disable_bounds_checks=True),
    )
    return f(x_flat, dsc, wp)


def _ffn_body(xe_ref, wg_ref, wu_ref, wd_ref, ws_ref, y_ref):
    h = pl.program_id(1)
    xe = xe_ref[...]
    wg = wg_ref[0]
    wu = wu_ref[0]
    wd = wd_ref[0]

    dn = (((1,), (1,)), ((), ()))
    g = lax.dot_general(xe, wg, dn, preferred_element_type=jnp.float32,
                        precision=lax.Precision.DEFAULT)
    u = lax.dot_general(xe, wu, dn, preferred_element_type=jnp.float32,
                        precision=lax.Precision.DEFAULT)
    act = (g / (1.0 + jnp.exp(-g))) * u
    y = lax.dot_general(act, wd, dn, preferred_element_type=jnp.float32,
                        precision=lax.Precision.DEFAULT)

    if NH == 1:
        y_ref[...] = y * ws_ref[...]
    else:
        @pl.when(h == 0)
        def _init():
            y_ref[...] = y

        @pl.when((h > 0) & (h < NH - 1))
        def _acc():
            y_ref[...] += y

        @pl.when(h == NH - 1)
        def _scale():
            y_ref[...] = (y_ref[...] + y) * ws_ref[...]


def _ffn(xe, Wg, Wu, Wd, ws):
    return pl.pallas_call(
        _ffn_body,
        grid=(E, NH),
        in_specs=[
            pl.BlockSpec((CAP, DIM), lambda e, h: (e, 0)),
            pl.BlockSpec((1, HB, DIM), lambda e, h: (e, h, 0)),
            pl.BlockSpec((1, HB, DIM), lambda e, h: (e, h, 0)),
            pl.BlockSpec((1, DIM, HB), lambda e, h: (e, 0, h)),
            pl.BlockSpec((CAP, 1), lambda e, h: (e, 0)),
        ],
        out_specs=pl.BlockSpec((CAP, DIM), lambda e, h: (e, 0)),
        out_shape=jax.ShapeDtypeStruct((SLOTS, DIM), jnp.float32),
        compiler_params=pltpu.CompilerParams(
            dimension_semantics=("parallel", "arbitrary")),
    )(xe, Wg, Wu, Wd, ws)


def _combine_body(y_hbm, dg_hbm, out_hbm, idx_v, rows_v, sem):
    wid = lax.axis_index("s") * NC + lax.axis_index("c")
    base = wid * COMB_CHUNK
    pltpu.sync_copy(dg_hbm.at[pl.ds(base, COMB_CHUNK)], idx_v)
    pltpu.async_copy(y_hbm.at[idx_v], rows_v, sem).wait()
    pltpu.sync_copy(rows_v, out_hbm.at[pl.ds(base, COMB_CHUNK)])


def _combine(y, dg):
    mesh = plsc.VectorSubcoreMesh(core_axis_name="c", subcore_axis_name="s")
    f = pl.kernel(
        _combine_body,
        out_type=jax.ShapeDtypeStruct((N_TOK, DIM), jnp.float32),
        mesh=mesh,
        scratch_types=[
            pltpu.VMEM((COMB_CHUNK,), jnp.int32),
            pltpu.VMEM((COMB_CHUNK, DIM), jnp.float32),
            pltpu.SemaphoreType.DMA,
        ],
        compiler_params=pltpu.CompilerParams(needs_layout_passes=False,
                                             disable_bounds_checks=True),
    )
    return f(y, dg)


@jax.jit
def kernel(x, gate_W, Wg, Wu, Wd):
    B, S, D = x.shape
    x_flat = x.reshape(-1, D)
    dsc, dg, wp = _router(x_flat, gate_W)
    dsc = dsc.reshape(-1)
    dg = dg.reshape(-1)
    wp = wp.reshape(-1)
    xe, ws = _dispatch(x_flat, dsc, wp)
    y = _ffn(xe, Wg, Wu, Wd, ws.reshape(SLOTS, 1))
    out = _combine(y, dg)
    return out.reshape(B, S, D)


# speedup vs baseline: 1.0543x; 1.0543x over previous
import functools

import jax
import jax.numpy as jnp
from jax import lax
from jax.experimental import pallas as pl
from jax.experimental.pallas import tpu as pltpu
from jax.experimental.pallas import tpu_sc as plsc

DIM = 1024
HIDDEN = 2048
E = 16
N_TOK = 2048
CAP = 160
SLOTS = E * CAP

NC = 2
NS = 16
NW = NC * NS
DISP_CHUNK = SLOTS // NW
COMB_CHUNK = N_TOK // NW

HB = 1024
NH = HIDDEN // HB


def _router_body(x_ref, gw_ref, dsc_ref, dg_ref, wp_ref):
    x = x_ref[...]
    gw = gw_ref[...]
    logits = lax.dot_general(
        gw, x, (((1,), (1,)), ((), ())),
        preferred_element_type=jnp.float32,
        precision=lax.Precision.DEFAULT)

    m = jnp.max(logits, axis=0, keepdims=True)
    ex = jnp.exp(logits - m)
    s = jnp.sum(ex, axis=0, keepdims=True)
    p = 1.0 / s
    w = p / (p + 1e-9)

    erow = lax.broadcasted_iota(jnp.int32, (E, N_TOK), 0)
    ids = jnp.min(jnp.where(logits == m, erow, E), axis=0, keepdims=True)

    onehot = (erow == ids).astype(jnp.float32)

    cols = lax.broadcasted_iota(jnp.int32, (E, N_TOK), 1)
    cum = onehot
    k = 1
    while k < N_TOK:
        rolled = pltpu.roll(cum, k, 1)
        cum = cum + jnp.where(cols >= k, rolled, 0.0)
        k *= 2

    slot = (jnp.sum(onehot * cum, axis=0, keepdims=True) - 1.0).astype(jnp.int32)

    cnt = cum[:, N_TOK - 1:N_TOK].astype(jnp.int32)
    ecol = lax.broadcasted_iota(jnp.int32, (E, 1), 0)
    e_star = jnp.min(jnp.where(cnt < CAP, ecol, E), axis=0, keepdims=True)
    cnt_star = jnp.sum(jnp.where(ecol == e_star, cnt, 0), axis=0, keepdims=True)
    sentinel = e_star * CAP + cnt_star

    valid = slot < CAP
    dest = ids * CAP + slot
    dsc_ref[...] = jnp.where(valid, dest, SLOTS)
    dg_ref[...] = jnp.where(valid, dest, jnp.broadcast_to(sentinel, (1, N_TOK)))
    wp_ref[...] = w


def _router(x_flat, gate_W):
    return pl.pallas_call(
        _router_body,
        out_shape=(
            jax.ShapeDtypeStruct((1, N_TOK), jnp.int32),
            jax.ShapeDtypeStruct((1, N_TOK), jnp.int32),
            jax.ShapeDtypeStruct((1, N_TOK), jnp.float32),
        ),
    )(x_flat, gate_W)


def _dispatch_body(x_hbm, dsc_hbm, wp_hbm, xe_hbm, ws_hbm,
                   dsc_v, wp_v, tidx_v, wslot_v, idx_v, wsb_v, rows_v, sem):
    wid = lax.axis_index("s") * NC + lax.axis_index("c")
    pltpu.sync_copy(dsc_hbm, dsc_v)
    pltpu.sync_copy(wp_hbm, wp_v)

    zero_f = jnp.zeros((16,), jnp.float32)
    lane = lax.iota(jnp.int32, 16)

    @plsc.parallel_loop(0, SLOTS // 16, unroll=8)
    def _init(i):
        off = pl.multiple_of(i * 16, 16)
        p = lane + i * 16
        tidx_v[pl.ds(off, 16)] = jnp.where(p < N_TOK, p, p - N_TOK)
        wslot_v[pl.ds(off, 16)] = zero_f

    @plsc.parallel_loop(0, N_TOK // 16, unroll=8)
    def _scatter(i):
        off = pl.multiple_of(i * 16, 16)
        d = dsc_v[pl.ds(off, 16)]
        mvalid = d < SLOTS
        dc = jnp.minimum(d, SLOTS - 1)
        plsc.store_scatter(tidx_v, [dc], lane + i * 16, mask=mvalid)
        wv = wp_v[pl.ds(off, 16)]
        plsc.store_scatter(wslot_v, [dc], wv, mask=mvalid)

    base = wid * DISP_CHUNK
    for j in range(DISP_CHUNK // 16):
        src = pl.multiple_of(base + j * 16, 16)
        idx_v[pl.ds(j * 16, 16)] = tidx_v[pl.ds(src, 16)]
        wsb_v[pl.ds(j * 16, 16)] = wslot_v[pl.ds(src, 16)]

    pltpu.async_copy(x_hbm.at[idx_v], rows_v, sem).wait()
    pltpu.sync_copy(rows_v, xe_hbm.at[pl.ds(base, DISP_CHUNK)])
    pltpu.sync_copy(wsb_v, ws_hbm.at[pl.ds(base, DISP_CHUNK)])


def _dispatch(x_flat, dsc, wp):
    mesh = plsc.VectorSubcoreMesh(core_axis_name="c", subcore_axis_name="s")
    f = pl.kernel(
        _dispatch_body,
        out_type=(
            jax.ShapeDtypeStruct((SLOTS, DIM), jnp.float32),
            jax.ShapeDtypeStruct((SLOTS,), jnp.float32),
        ),
        mesh=mesh,
        scratch_types=[
            pltpu.VMEM((N_TOK,), jnp.int32),
            pltpu.VMEM((N_TOK,), jnp.float32),
            pltpu.VMEM((SLOTS,), jnp.int32),
            pltpu.VMEM((SLOTS,), jnp.float32),
            pltpu.VMEM((DISP_CHUNK,), jnp.int32),
            pltpu.VMEM((DISP_CHUNK,), jnp.float32),
            pltpu.VMEM((DISP_CHUNK, DIM), jnp.float32),
            pltpu.SemaphoreType.DMA,
        ],
        compiler_params=pltpu.CompilerParams(needs_layout_passes=False,
                                             disable_bounds_checks=True),
    )
    return f(x_flat, dsc, wp)


def _ffn_body(xe_ref, wg_ref, wu_ref, wd_ref, ws_ref, y_ref):
    h = pl.program_id(1)
    xe = xe_ref[...]
    wg = wg_ref[0]
    wu = wu_ref[0]
    wd = wd_ref[0]

    dn = (((1,), (1,)), ((), ()))
    g = lax.dot_general(xe, wg, dn, preferred_element_type=jnp.float32,
                        precision=lax.Precision.DEFAULT)
    u = lax.dot_general(xe, wu, dn, preferred_element_type=jnp.float32,
                        precision=lax.Precision.DEFAULT)
    act = (g / (1.0 + jnp.exp(-g))) * u
    y = lax.dot_general(act, wd, dn, preferred_element_type=jnp.float32,
                        precision=lax.Precision.DEFAULT)

    if NH == 1:
        y_ref[...] = y * ws_ref[...]
    else:
        @pl.when(h == 0)
        def _init():
            y_ref[...] = y

        @pl.when((h > 0) & (h < NH - 1))
        def _acc():
            y_ref[...] += y

        @pl.when(h == NH - 1)
        def _scale():
            y_ref[...] = (y_ref[...] + y) * ws_ref[...]


def _ffn(xe, Wg, Wu, Wd, ws):
    return pl.pallas_call(
        _ffn_body,
        grid=(E, NH),
        in_specs=[
            pl.BlockSpec((CAP, DIM), lambda e, h: (e, 0)),
            pl.BlockSpec((1, HB, DIM), lambda e, h: (e, h, 0)),
            pl.BlockSpec((1, HB, DIM), lambda e, h: (e, h, 0)),
            pl.BlockSpec((1, DIM, HB), lambda e, h: (e, 0, h)),
            pl.BlockSpec((CAP, 1), lambda e, h: (e, 0)),
        ],
        out_specs=pl.BlockSpec((CAP, DIM), lambda e, h: (e, 0)),
        out_shape=jax.ShapeDtypeStruct((SLOTS, DIM), jnp.float32),
        compiler_params=pltpu.CompilerParams(
            dimension_semantics=("arbitrary", "arbitrary")),
    )(xe, Wg, Wu, Wd, ws)


def _combine_body(y_hbm, dg_hbm, out_hbm, idx_v, rows_v, sem):
    wid = lax.axis_index("s") * NC + lax.axis_index("c")
    base = wid * COMB_CHUNK
    pltpu.sync_copy(dg_hbm.at[pl.ds(base, COMB_CHUNK)], idx_v)
    pltpu.async_copy(y_hbm.at[idx_v], rows_v, sem).wait()
    pltpu.sync_copy(rows_v, out_hbm.at[pl.ds(base, COMB_CHUNK)])


def _combine(y, dg):
    mesh = plsc.VectorSubcoreMesh(core_axis_name="c", subcore_axis_name="s")
    f = pl.kernel(
        _combine_body,
        out_type=jax.ShapeDtypeStruct((N_TOK, DIM), jnp.float32),
        mesh=mesh,
        scratch_types=[
            pltpu.VMEM((COMB_CHUNK,), jnp.int32),
            pltpu.VMEM((COMB_CHUNK, DIM), jnp.float32),
            pltpu.SemaphoreType.DMA,
        ],
        compiler_params=pltpu.CompilerParams(needs_layout_passes=False,
                                             disable_bounds_checks=True),
    )
    return f(y, dg)


@jax.jit
def kernel(x, gate_W, Wg, Wu, Wd):
    B, S, D = x.shape
    x_flat = x.reshape(-1, D)
    dsc, dg, wp = _router(x_flat, gate_W)
    dsc = dsc.reshape(-1)
    dg = dg.reshape(-1)
    wp = wp.reshape(-1)
    xe, ws = _dispatch(x_flat, dsc, wp)
    y = _ffn(xe, Wg, Wu, Wd, ws.reshape(SLOTS, 1))
    out = _combine(y, dg)
    return out.reshape(B, S, D)
